# Initial kernel scaffold; baseline (speedup 1.0000x reference)
import functools

import jax
import jax.numpy as jnp
from jax import lax
from jax.experimental import pallas as pl
from jax.experimental.pallas import tpu as pltpu
from jax.experimental.pallas import tpu_sc as plsc

N = 10000
E = 320000
D = 128
NPAD = 10240

NC = 2
NS = 16
L = 16
NW = NC * NS
EW = E // NW
C = 80
NCH = EW // C
RPW = N // NS
ZR = 125

_mesh = plsc.VectorSubcoreMesh(core_axis_name="c", subcore_axis_name="s")


@functools.partial(
    pl.kernel,
    out_type=jax.ShapeDtypeStruct((NW, NPAD), jnp.float32),
    mesh=_mesh,
    scratch_types=[
        pltpu.VMEM((NPAD,), jnp.float32),
        pltpu.VMEM((EW,), jnp.int32),
        pltpu.VMEM((EW,), jnp.float32),
    ],
)
def _deg(row_hbm, attr_hbm, out_hbm, acc, rows, attrs):
    c = lax.axis_index("c")
    s = lax.axis_index("s")
    gw = c * NS + s
    base = gw * EW
    pltpu.sync_copy(row_hbm.at[pl.ds(base, EW)], rows)
    pltpu.sync_copy(attr_hbm.at[pl.ds(base, EW)], attrs)

    def zero(i, carry):
        acc[pl.ds(i * L, L)] = jnp.zeros((L,), jnp.float32)
        return carry

    lax.fori_loop(0, NPAD // L, zero, 0)

    def body(i, carry):
        r = rows[pl.ds(i * L, L)]
        a = attrs[pl.ds(i * L, L)]
        plsc.addupdate_scatter(acc, [r], a)
        return carry

    lax.fori_loop(0, EW // L, body, 0)
    pltpu.sync_copy(acc, out_hbm.at[gw])


def _dinv_body(degp_ref, dinv_ref):
    deg = jnp.sum(degp_ref[...], axis=0)
    r = lax.rsqrt(jnp.maximum(deg, 1e-12))
    dinv_ref[...] = jnp.where(deg > 0, r, 0.0)


def _dinv(degp):
    return pl.pallas_call(
        _dinv_body,
        out_shape=jax.ShapeDtypeStruct((NPAD // 128, 128), jnp.float32),
    )(degp.reshape(NW, NPAD // 128, 128))


@functools.partial(
    pl.kernel,
    out_type=jax.ShapeDtypeStruct((E,), jnp.float32),
    mesh=_mesh,
    scratch_types=[
        pltpu.VMEM((NPAD,), jnp.float32),
        pltpu.VMEM((EW,), jnp.int32),
        pltpu.VMEM((EW,), jnp.int32),
        pltpu.VMEM((EW,), jnp.float32),
        pltpu.VMEM((EW,), jnp.float32),
    ],
)
def _lap(row_hbm, col_hbm, attr_hbm, dinv_hbm, lap_hbm, dinv_v, rows, cols, attrs, lap_v):
    c = lax.axis_index("c")
    s = lax.axis_index("s")
    gw = c * NS + s
    base = gw * EW
    pltpu.sync_copy(dinv_hbm, dinv_v)
    pltpu.sync_copy(row_hbm.at[pl.ds(base, EW)], rows)
    pltpu.sync_copy(col_hbm.at[pl.ds(base, EW)], cols)
    pltpu.sync_copy(attr_hbm.at[pl.ds(base, EW)], attrs)

    def body(i, carry):
        sl = pl.ds(i * L, L)
        dr = plsc.load_gather(dinv_v, [rows[sl]])
        dc = plsc.load_gather(dinv_v, [cols[sl]])
        lap_v[sl] = -(dr * attrs[sl] * dc)
        return carry

    lax.fori_loop(0, EW // L, body, 0)
    pltpu.sync_copy(lap_v, lap_hbm.at[pl.ds(base, EW)])


@functools.partial(
    pl.kernel,
    out_type=jax.ShapeDtypeStruct((NC, N, D), jnp.float32),
    mesh=_mesh,
    scratch_types=[
        pltpu.VMEM_SHARED((N, D), jnp.float32),
        pltpu.VMEM((C,), jnp.int32),
        pltpu.VMEM((C,), jnp.int32),
        pltpu.VMEM((C,), jnp.float32),
        pltpu.VMEM((C, D), jnp.float32),
        pltpu.VMEM((ZR, D), jnp.float32),
        pltpu.SemaphoreType.DMA,
    ],
)
def _spmm(m_hbm, col_hbm, row_hbm, lap_hbm, out_hbm, acc, colb, rowb, lapb, rbuf, zbuf, gsem):
    c = lax.axis_index("c")
    s = lax.axis_index("s")
    gw = c * NS + s

    def zfill(i, carry):
        for j in range(D // L):
            zbuf[i, pl.ds(j * L, L)] = jnp.zeros((L,), jnp.float32)
        return carry

    lax.fori_loop(0, ZR, zfill, 0)
    for k in range(RPW // ZR):
        pltpu.sync_copy(zbuf, acc.at[pl.ds(s * RPW + k * ZR, ZR), :])
    plsc.subcore_barrier()

    def chunk(i, carry):
        base = gw * EW + i * C
        pltpu.sync_copy(col_hbm.at[pl.ds(base, C)], colb)
        pltpu.sync_copy(row_hbm.at[pl.ds(base, C)], rowb)
        pltpu.sync_copy(lap_hbm.at[pl.ds(base, C)], lapb)
        pltpu.async_copy(m_hbm.at[colb], rbuf, gsem).wait()

        def edge(e, carry2):
            lv = plsc.load_gather(lapb, [jnp.full((L,), 0, jnp.int32) + e])
            for j in range(D // L):
                sl = pl.ds(j * L, L)
                rbuf[e, sl] = rbuf[e, sl] * lv
            return carry2

        lax.fori_loop(0, C, edge, 0)
        pltpu.sync_copy(rbuf, acc.at[rowb], add=True)
        return carry

    lax.fori_loop(0, NCH, chunk, 0)
    plsc.subcore_barrier()
    pltpu.sync_copy(acc.at[pl.ds(s * RPW, RPW), :], out_hbm.at[c, pl.ds(s * RPW, RPW), :])


_RB = 400


def _combine_body(a, b, p0_ref, p1_ref, prev_ref, out_ref):
    out_ref[...] = a * (p0_ref[...] + p1_ref[...]) + b * prev_ref[...]


def _combine(p0, p1, prev, a, b):
    grid = N // _RB
    bs = pl.BlockSpec((_RB, D), lambda i: (i, 0))
    return pl.pallas_call(
        functools.partial(_combine_body, a, b),
        grid=(grid,),
        in_specs=[bs, bs, bs],
        out_specs=bs,
        out_shape=jax.ShapeDtypeStruct((N, D), jnp.float32),
    )(p0, p1, prev)


def _final_body(x_ref, t1_ref, t2_ref, p0_ref, p1_ref, w_ref, b_ref, out_ref):
    t3 = 2.0 * (p0_ref[...] + p1_ref[...]) - t1_ref[...]
    w = w_ref[...]
    acc = jnp.dot(x_ref[...], w[0], preferred_element_type=jnp.float32)
    acc += jnp.dot(t1_ref[...], w[1], preferred_element_type=jnp.float32)
    acc += jnp.dot(t2_ref[...], w[2], preferred_element_type=jnp.float32)
    acc += jnp.dot(t3, w[3], preferred_element_type=jnp.float32)
    out_ref[...] = acc + b_ref[...]


def _final(x, t1, t2, p0, p1, weight, bias):
    grid = N // _RB
    bs = pl.BlockSpec((_RB, D), lambda i: (i, 0))
    return pl.pallas_call(
        _final_body,
        grid=(grid,),
        in_specs=[
            bs, bs, bs, bs, bs,
            pl.BlockSpec((4, D, D), lambda i: (0, 0, 0)),
            pl.BlockSpec((1, D), lambda i: (0, 0)),
        ],
        out_specs=bs,
        out_shape=jax.ShapeDtypeStruct((N, D), jnp.float32),
    )(x, t1, t2, p0, p1, weight, bias.reshape(1, D))


def kernel(x, edge_index, edge_attr, weight, bias):
    row = edge_index[0]
    col = edge_index[1]
    degp = _deg(row, edge_attr)
    dinv = _dinv(degp).reshape(NPAD)
    lap = _lap(row, col, edge_attr, dinv)
    p = _spmm(x, col, row, lap)
    t1 = _combine(p[0], p[1], x, 1.0, 0.0)
    p = _spmm(t1, col, row, lap)
    t2 = _combine(p[0], p[1], x, 2.0, -1.0)
    p = _spmm(t2, col, row, lap)
    return _final(x, t1, t2, p[0], p[1], weight, bias)


# speedup vs baseline: 6.0906x; 6.0906x over previous
import functools

import jax
import jax.numpy as jnp
from jax import lax
from jax.experimental import pallas as pl
from jax.experimental.pallas import tpu as pltpu
from jax.experimental.pallas import tpu_sc as plsc

N = 10000
E = 320000
D = 128
NPAD = 10240

NC = 2
NS = 16
L = 16
NW = NC * NS
EW = E // NW
C = 80
NCH = EW // C
RPW = NPAD // NS
ZR = 128

_mesh = plsc.VectorSubcoreMesh(core_axis_name="c", subcore_axis_name="s")


@functools.partial(
    pl.kernel,
    out_type=jax.ShapeDtypeStruct((NW * NPAD,), jnp.float32),
    mesh=_mesh,
    compiler_params=pltpu.CompilerParams(needs_layout_passes=False),
    scratch_types=[
        pltpu.VMEM((NPAD,), jnp.float32),
        pltpu.VMEM((EW,), jnp.int32),
        pltpu.VMEM((EW,), jnp.float32),
    ],
)
def _deg(row_hbm, attr_hbm, out_hbm, acc, rows, attrs):
    c = lax.axis_index("c")
    s = lax.axis_index("s")
    gw = c * NS + s
    base = gw * EW
    pltpu.sync_copy(row_hbm.at[pl.ds(base, EW)], rows)
    pltpu.sync_copy(attr_hbm.at[pl.ds(base, EW)], attrs)

    def zero(i, carry):
        acc[pl.ds(i * L, L)] = jnp.zeros((L,), jnp.float32)
        return carry

    lax.fori_loop(0, NPAD // L, zero, 0)

    def body(i, carry):
        r = rows[pl.ds(i * L, L)]
        a = attrs[pl.ds(i * L, L)]
        plsc.addupdate_scatter(acc, [r], a)
        return carry

    lax.fori_loop(0, EW // L, body, 0)
    pltpu.sync_copy(acc, out_hbm.at[pl.ds(gw * NPAD, NPAD)])


def _dinv_body(degp_ref, dinv_ref):
    deg = jnp.sum(degp_ref[...], axis=0)
    r = lax.rsqrt(jnp.maximum(deg, 1e-12))
    dinv_ref[...] = jnp.where(deg > 0, r, 0.0)


def _dinv(degp):
    return pl.pallas_call(
        _dinv_body,
        out_shape=jax.ShapeDtypeStruct((NPAD // 128, 128), jnp.float32),
    )(degp.reshape(NW, NPAD // 128, 128))


@functools.partial(
    pl.kernel,
    out_type=jax.ShapeDtypeStruct((E,), jnp.float32),
    mesh=_mesh,
    compiler_params=pltpu.CompilerParams(needs_layout_passes=False),
    scratch_types=[
        pltpu.VMEM((NPAD,), jnp.float32),
        pltpu.VMEM((EW,), jnp.int32),
        pltpu.VMEM((EW,), jnp.int32),
        pltpu.VMEM((EW,), jnp.float32),
        pltpu.VMEM((EW,), jnp.float32),
    ],
)
def _lap(row_hbm, col_hbm, attr_hbm, dinv_hbm, lap_hbm, dinv_v, rows, cols, attrs, lap_v):
    c = lax.axis_index("c")
    s = lax.axis_index("s")
    gw = c * NS + s
    base = gw * EW
    pltpu.sync_copy(dinv_hbm, dinv_v)
    pltpu.sync_copy(row_hbm.at[pl.ds(base, EW)], rows)
    pltpu.sync_copy(col_hbm.at[pl.ds(base, EW)], cols)
    pltpu.sync_copy(attr_hbm.at[pl.ds(base, EW)], attrs)

    def body(i, carry):
        sl = pl.ds(i * L, L)
        dr = plsc.load_gather(dinv_v, [rows[sl]])
        dc = plsc.load_gather(dinv_v, [cols[sl]])
        lap_v[sl] = -(dr * attrs[sl] * dc)
        return carry

    lax.fori_loop(0, EW // L, body, 0)
    pltpu.sync_copy(lap_v, lap_hbm.at[pl.ds(base, EW)])


@functools.partial(
    pl.kernel,
    out_type=jax.ShapeDtypeStruct((NC, NPAD, D), jnp.float32),
    mesh=_mesh,
    compiler_params=pltpu.CompilerParams(needs_layout_passes=False),
    scratch_types=[
        pltpu.VMEM_SHARED((NPAD, D), jnp.float32),
        pltpu.VMEM((C,), jnp.int32),
        pltpu.VMEM((C,), jnp.int32),
        pltpu.VMEM((C,), jnp.float32),
        pltpu.VMEM((C, D), jnp.float32),
        pltpu.VMEM((ZR, D), jnp.float32),
        pltpu.SemaphoreType.DMA,
    ],
)
def _spmm(m_hbm, col_hbm, row_hbm, lap_hbm, out_hbm, acc, colb, rowb, lapb, rbuf, zbuf, gsem):
    c = lax.axis_index("c")
    s = lax.axis_index("s")
    gw = c * NS + s

    def zfill(i, carry):
        for j in range(D // L):
            zbuf[i, pl.ds(j * L, L)] = jnp.zeros((L,), jnp.float32)
        return carry

    lax.fori_loop(0, ZR, zfill, 0)
    for k in range(RPW // ZR):
        pltpu.sync_copy(zbuf, acc.at[pl.ds(s * RPW + k * ZR, ZR), :])
    plsc.subcore_barrier()

    def chunk(i, carry):
        base = gw * EW + i * C
        pltpu.sync_copy(col_hbm.at[pl.ds(base, C)], colb)
        pltpu.sync_copy(row_hbm.at[pl.ds(base, C)], rowb)
        pltpu.sync_copy(lap_hbm.at[pl.ds(base, C)], lapb)
        pltpu.async_copy(m_hbm.at[colb], rbuf, gsem).wait()

        def edge(e, carry2):
            lv = plsc.load_gather(lapb, [jnp.full((L,), 0, jnp.int32) + e])
            for j in range(D // L):
                sl = pl.ds(j * L, L)
                rbuf[e, sl] = rbuf[e, sl] * lv
            return carry2

        lax.fori_loop(0, C, edge, 0)
        pltpu.sync_copy(rbuf, acc.at[rowb], add=True)
        return carry

    lax.fori_loop(0, NCH, chunk, 0)
    plsc.subcore_barrier()
    pltpu.sync_copy(acc.at[pl.ds(s * RPW, RPW), :], out_hbm.at[c, pl.ds(s * RPW, RPW), :])


_RB = 400


def _combine_body(a, b, p0_ref, p1_ref, prev_ref, out_ref):
    out_ref[...] = a * (p0_ref[...] + p1_ref[...]) + b * prev_ref[...]


def _combine(p0, p1, prev, a, b):
    grid = NPAD // 512
    bs = pl.BlockSpec((512, D), lambda i: (i, 0))
    return pl.pallas_call(
        functools.partial(_combine_body, a, b),
        grid=(grid,),
        in_specs=[bs, bs, bs],
        out_specs=bs,
        out_shape=jax.ShapeDtypeStruct((NPAD, D), jnp.float32),
    )(p0, p1, prev)


def _final_body(x_ref, t1_ref, t2_ref, p0_ref, p1_ref, w_ref, b_ref, out_ref):
    t3 = 2.0 * (p0_ref[...] + p1_ref[...]) - t1_ref[...]
    w = w_ref[...]
    acc = jnp.dot(x_ref[...], w[0], preferred_element_type=jnp.float32)
    acc += jnp.dot(t1_ref[...], w[1], preferred_element_type=jnp.float32)
    acc += jnp.dot(t2_ref[...], w[2], preferred_element_type=jnp.float32)
    acc += jnp.dot(t3, w[3], preferred_element_type=jnp.float32)
    out_ref[...] = acc + b_ref[...]


def _final(x, t1, t2, p0, p1, weight, bias):
    grid = N // _RB
    bs = pl.BlockSpec((_RB, D), lambda i: (i, 0))
    return pl.pallas_call(
        _final_body,
        grid=(grid,),
        in_specs=[
            bs, bs, bs, bs, bs,
            pl.BlockSpec((4, D, D), lambda i: (0, 0, 0)),
            pl.BlockSpec((1, D), lambda i: (0, 0)),
        ],
        out_specs=bs,
        out_shape=jax.ShapeDtypeStruct((N, D), jnp.float32),
    )(x, t1, t2, p0, p1, weight, bias.reshape(1, D))


def kernel(x, edge_index, edge_attr, weight, bias):
    row = edge_index[0]
    col = edge_index[1]
    xp = jnp.pad(x, ((0, NPAD - N), (0, 0)))
    degp = _deg(row, edge_attr)
    dinv = _dinv(degp.reshape(NW, NPAD)).reshape(NPAD)
    lap = _lap(row, col, edge_attr, dinv)
    p = _spmm(xp, col, row, lap)
    t1 = _combine(p[0], p[1], xp, 1.0, 0.0)
    p = _spmm(t1, col, row, lap)
    t2 = _combine(p[0], p[1], xp, 2.0, -1.0)
    p = _spmm(t2, col, row, lap)
    return _final(xp, t1, t2, p[0], p[1], weight, bias)


# speedup vs baseline: 14.1802x; 2.3282x over previous
import functools

import jax
import jax.numpy as jnp
from jax import lax
from jax.experimental import pallas as pl
from jax.experimental.pallas import tpu as pltpu
from jax.experimental.pallas import tpu_sc as plsc

N = 10000
E = 320000
D = 128
NPAD = 10240

NC = 2
NS = 16
L = 16
NW = NC * NS
EW = E // NW
C = 80
NCH = EW // C
RPW = NPAD // NS
ZR = 128

_mesh = plsc.VectorSubcoreMesh(core_axis_name="c", subcore_axis_name="s")


@functools.partial(
    pl.kernel,
    out_type=jax.ShapeDtypeStruct((NW * NPAD,), jnp.float32),
    mesh=_mesh,
    compiler_params=pltpu.CompilerParams(needs_layout_passes=False),
    scratch_types=[
        pltpu.VMEM((NPAD,), jnp.float32),
        pltpu.VMEM((EW,), jnp.int32),
        pltpu.VMEM((EW,), jnp.float32),
    ],
)
def _deg(row_hbm, attr_hbm, out_hbm, acc, rows, attrs):
    c = lax.axis_index("c")
    s = lax.axis_index("s")
    gw = c * NS + s
    base = gw * EW
    pltpu.sync_copy(row_hbm.at[pl.ds(base, EW)], rows)
    pltpu.sync_copy(attr_hbm.at[pl.ds(base, EW)], attrs)

    def zero(i, carry):
        acc[pl.ds(i * L, L)] = jnp.zeros((L,), jnp.float32)
        return carry

    lax.fori_loop(0, NPAD // L, zero, 0)

    def body(i, carry):
        r = rows[pl.ds(i * L, L)]
        a = attrs[pl.ds(i * L, L)]
        plsc.addupdate_scatter(acc, [r], a)
        return carry

    lax.fori_loop(0, EW // L, body, 0)
    pltpu.sync_copy(acc, out_hbm.at[pl.ds(gw * NPAD, NPAD)])


def _dinv_body(degp_ref, dinv_ref):
    deg = jnp.sum(degp_ref[...], axis=0)
    r = lax.rsqrt(jnp.maximum(deg, 1e-12))
    dinv_ref[...] = jnp.where(deg > 0, r, 0.0)


def _dinv(degp):
    return pl.pallas_call(
        _dinv_body,
        out_shape=jax.ShapeDtypeStruct((NPAD // 128, 128), jnp.float32),
    )(degp.reshape(NW, NPAD // 128, 128))


@functools.partial(
    pl.kernel,
    out_type=jax.ShapeDtypeStruct((E,), jnp.float32),
    mesh=_mesh,
    compiler_params=pltpu.CompilerParams(needs_layout_passes=False),
    scratch_types=[
        pltpu.VMEM((NPAD,), jnp.float32),
        pltpu.VMEM((EW,), jnp.int32),
        pltpu.VMEM((EW,), jnp.int32),
        pltpu.VMEM((EW,), jnp.float32),
        pltpu.VMEM((EW,), jnp.float32),
    ],
)
def _lap(row_hbm, col_hbm, attr_hbm, dinv_hbm, lap_hbm, dinv_v, rows, cols, attrs, lap_v):
    c = lax.axis_index("c")
    s = lax.axis_index("s")
    gw = c * NS + s
    base = gw * EW
    pltpu.sync_copy(dinv_hbm, dinv_v)
    pltpu.sync_copy(row_hbm.at[pl.ds(base, EW)], rows)
    pltpu.sync_copy(col_hbm.at[pl.ds(base, EW)], cols)
    pltpu.sync_copy(attr_hbm.at[pl.ds(base, EW)], attrs)

    def body(i, carry):
        sl = pl.ds(i * L, L)
        dr = plsc.load_gather(dinv_v, [rows[sl]])
        dc = plsc.load_gather(dinv_v, [cols[sl]])
        lap_v[sl] = -(dr * attrs[sl] * dc)
        return carry

    lax.fori_loop(0, EW // L, body, 0)
    pltpu.sync_copy(lap_v, lap_hbm.at[pl.ds(base, EW)])


@functools.partial(
    pl.kernel,
    out_type=jax.ShapeDtypeStruct((NC, NPAD, D), jnp.float32),
    mesh=_mesh,
    compiler_params=pltpu.CompilerParams(needs_layout_passes=False),
    scratch_types=[
        pltpu.VMEM_SHARED((NPAD, D), jnp.float32),
        pltpu.VMEM((NCH, C), jnp.int32),
        pltpu.VMEM((C,), jnp.int32),
        pltpu.VMEM((C,), jnp.int32),
        pltpu.VMEM((C,), jnp.float32),
        pltpu.VMEM((C,), jnp.float32),
        pltpu.VMEM((C, D), jnp.float32),
        pltpu.VMEM((C, D), jnp.float32),
        pltpu.SemaphoreType.DMA,
        pltpu.SemaphoreType.DMA,
        pltpu.SemaphoreType.DMA,
        pltpu.SemaphoreType.DMA,
        pltpu.SemaphoreType.DMA,
        pltpu.SemaphoreType.DMA,
        pltpu.SemaphoreType.DMA,
        pltpu.SemaphoreType.DMA,
    ],
)
def _spmm(m_hbm, col3, row_hbm, lap_hbm, out_hbm, acc, colb2, rowp0, rowp1,
          lapp0, lapp1, rbuf0, rbuf1, gsem0, gsem1, ssem0, ssem1,
          psem0, psem1, qsem0, qsem1):
    c = lax.axis_index("c")
    s = lax.axis_index("s")
    gw = c * NS + s

    pltpu.sync_copy(col3.at[gw], colb2)

    def zfill(i, carry):
        for j in range(D // L):
            rbuf0[i, pl.ds(j * L, L)] = jnp.zeros((L,), jnp.float32)
        return carry

    lax.fori_loop(0, C, zfill, 0)
    for k in range(RPW // C):
        pltpu.sync_copy(rbuf0, acc.at[pl.ds(s * RPW + k * C, C), :])
    plsc.subcore_barrier()

    def g_copy(k, buf, sem):
        return pltpu.make_async_copy(m_hbm.at[colb2.at[k]], buf, sem)

    def s_copy(k, buf, rowp, sem):
        return pltpu.make_async_copy(buf, acc.at[rowp], sem)

    def r_copy(k, rowp, sem):
        return pltpu.make_async_copy(row_hbm.at[pl.ds(gw * EW + k * C, C)], rowp, sem)

    def l_copy(k, lapp, sem):
        return pltpu.make_async_copy(lap_hbm.at[pl.ds(gw * EW + k * C, C)], lapp, sem)

    def scale(buf, lapp):
        def edge(e, carry):
            lv = plsc.load_gather(lapp, [jnp.zeros((L,), jnp.int32) + e])
            for j in range(D // L):
                sl = pl.ds(j * L, L)
                buf[e, sl] = buf[e, sl] * lv
            return carry

        lax.fori_loop(0, C, edge, 0)

    r_copy(0, rowp0, psem0).start()
    l_copy(0, lapp0, qsem0).start()
    r_copy(1, rowp1, psem1).start()
    l_copy(1, lapp1, qsem1).start()
    g_copy(0, rbuf0, gsem0).start()
    g_copy(0, rbuf0, gsem0).wait()
    r_copy(0, rowp0, psem0).wait()
    l_copy(0, lapp0, qsem0).wait()
    g_copy(1, rbuf1, gsem1).start()
    scale(rbuf0, lapp0)
    s_copy(0, rbuf0, rowp0, ssem0).start(add=True)

    def outer(g, carry):
        k = 2 * g + 1
        g_copy(k, rbuf1, gsem1).wait()
        r_copy(k, rowp1, psem1).wait()
        l_copy(k, lapp1, qsem1).wait()
        s_copy(k - 1, rbuf0, rowp0, ssem0).wait()
        g_copy(k + 1, rbuf0, gsem0).start()
        r_copy(k + 1, rowp0, psem0).start()
        l_copy(k + 1, lapp0, qsem0).start()
        scale(rbuf1, lapp1)
        s_copy(k, rbuf1, rowp1, ssem1).start(add=True)
        k = 2 * g + 2
        g_copy(k, rbuf0, gsem0).wait()
        r_copy(k, rowp0, psem0).wait()
        l_copy(k, lapp0, qsem0).wait()
        s_copy(k - 1, rbuf1, rowp1, ssem1).wait()
        g_copy(k + 1, rbuf1, gsem1).start()
        r_copy(k + 1, rowp1, psem1).start()
        l_copy(k + 1, lapp1, qsem1).start()
        scale(rbuf0, lapp0)
        s_copy(k, rbuf0, rowp0, ssem0).start(add=True)
        return carry

    lax.fori_loop(0, (NCH - 3) // 2, outer, 0)

    k = NCH - 2
    g_copy(k, rbuf1, gsem1).wait()
    r_copy(k, rowp1, psem1).wait()
    l_copy(k, lapp1, qsem1).wait()
    s_copy(k - 1, rbuf0, rowp0, ssem0).wait()
    g_copy(k + 1, rbuf0, gsem0).start()
    r_copy(k + 1, rowp0, psem0).start()
    l_copy(k + 1, lapp0, qsem0).start()
    scale(rbuf1, lapp1)
    s_copy(k, rbuf1, rowp1, ssem1).start(add=True)

    k = NCH - 1
    g_copy(k, rbuf0, gsem0).wait()
    r_copy(k, rowp0, psem0).wait()
    l_copy(k, lapp0, qsem0).wait()
    s_copy(k - 1, rbuf1, rowp1, ssem1).wait()
    scale(rbuf0, lapp0)
    s_copy(k, rbuf0, rowp0, ssem0).start(add=True)
    s_copy(k, rbuf0, rowp0, ssem0).wait()

    plsc.subcore_barrier()
    pltpu.sync_copy(acc.at[pl.ds(s * RPW, RPW), :], out_hbm.at[c, pl.ds(s * RPW, RPW), :])


_RB = 400


def _combine_body(a, b, p0_ref, p1_ref, prev_ref, out_ref):
    out_ref[...] = a * (p0_ref[...] + p1_ref[...]) + b * prev_ref[...]


def _combine(p0, p1, prev, a, b):
    grid = NPAD // 512
    bs = pl.BlockSpec((512, D), lambda i: (i, 0))
    return pl.pallas_call(
        functools.partial(_combine_body, a, b),
        grid=(grid,),
        in_specs=[bs, bs, bs],
        out_specs=bs,
        out_shape=jax.ShapeDtypeStruct((NPAD, D), jnp.float32),
    )(p0, p1, prev)


def _final_body(x_ref, t1_ref, t2_ref, p0_ref, p1_ref, w_ref, b_ref, out_ref):
    t3 = 2.0 * (p0_ref[...] + p1_ref[...]) - t1_ref[...]
    w = w_ref[...]
    acc = jnp.dot(x_ref[...], w[0], preferred_element_type=jnp.float32)
    acc += jnp.dot(t1_ref[...], w[1], preferred_element_type=jnp.float32)
    acc += jnp.dot(t2_ref[...], w[2], preferred_element_type=jnp.float32)
    acc += jnp.dot(t3, w[3], preferred_element_type=jnp.float32)
    out_ref[...] = acc + b_ref[...]


def _final(x, t1, t2, p0, p1, weight, bias):
    grid = N // _RB
    bs = pl.BlockSpec((_RB, D), lambda i: (i, 0))
    return pl.pallas_call(
        _final_body,
        grid=(grid,),
        in_specs=[
            bs, bs, bs, bs, bs,
            pl.BlockSpec((4, D, D), lambda i: (0, 0, 0)),
            pl.BlockSpec((1, D), lambda i: (0, 0)),
        ],
        out_specs=bs,
        out_shape=jax.ShapeDtypeStruct((N, D), jnp.float32),
    )(x, t1, t2, p0, p1, weight, bias.reshape(1, D))


def kernel(x, edge_index, edge_attr, weight, bias):
    row = edge_index[0]
    col = edge_index[1]
    xp = jnp.pad(x, ((0, NPAD - N), (0, 0)))
    degp = _deg(row, edge_attr)
    dinv = _dinv(degp.reshape(NW, NPAD)).reshape(NPAD)
    lap = _lap(row, col, edge_attr, dinv)
    col3 = col.reshape(NW, NCH, C)
    p = _spmm(xp, col3, row, lap)
    t1 = _combine(p[0], p[1], xp, 1.0, 0.0)
    p = _spmm(t1, col3, row, lap)
    t2 = _combine(p[0], p[1], xp, 2.0, -1.0)
    p = _spmm(t2, col3, row, lap)
    return _final(xp, t1, t2, p[0], p[1], weight, bias)


# speedup vs baseline: 16.2935x; 1.1490x over previous
import functools

import jax
import jax.numpy as jnp
from jax import lax
from jax.experimental import pallas as pl
from jax.experimental.pallas import tpu as pltpu
from jax.experimental.pallas import tpu_sc as plsc

N = 10000
E = 320000
D = 128
NPAD = 10240

NC = 2
NS = 16
L = 16
NW = NC * NS
EW = E // NW
C = 80
NCH = EW // C
RPW = NPAD // NS
ZR = 128

_mesh = plsc.VectorSubcoreMesh(core_axis_name="c", subcore_axis_name="s")


@functools.partial(
    pl.kernel,
    out_type=jax.ShapeDtypeStruct((NW * NPAD,), jnp.float32),
    mesh=_mesh,
    compiler_params=pltpu.CompilerParams(needs_layout_passes=False),
    scratch_types=[
        pltpu.VMEM((NPAD,), jnp.float32),
        pltpu.VMEM((EW,), jnp.int32),
        pltpu.VMEM((EW,), jnp.float32),
    ],
)
def _deg(row_hbm, attr_hbm, out_hbm, acc, rows, attrs):
    c = lax.axis_index("c")
    s = lax.axis_index("s")
    gw = c * NS + s
    base = gw * EW
    pltpu.sync_copy(row_hbm.at[pl.ds(base, EW)], rows)
    pltpu.sync_copy(attr_hbm.at[pl.ds(base, EW)], attrs)

    def zero(i, carry):
        acc[pl.ds(i * L, L)] = jnp.zeros((L,), jnp.float32)
        return carry

    lax.fori_loop(0, NPAD // L, zero, 0)

    def body(i, carry):
        r = rows[pl.ds(i * L, L)]
        a = attrs[pl.ds(i * L, L)]
        plsc.addupdate_scatter(acc, [r], a)
        return carry

    lax.fori_loop(0, EW // L, body, 0)
    pltpu.sync_copy(acc, out_hbm.at[pl.ds(gw * NPAD, NPAD)])


def _dinv_body(degp_ref, dinv_ref):
    deg = jnp.sum(degp_ref[...], axis=0)
    r = lax.rsqrt(jnp.maximum(deg, 1e-12))
    dinv_ref[...] = jnp.where(deg > 0, r, 0.0)


def _dinv(degp):
    return pl.pallas_call(
        _dinv_body,
        out_shape=jax.ShapeDtypeStruct((NPAD // 128, 128), jnp.float32),
    )(degp.reshape(NW, NPAD // 128, 128))


@functools.partial(
    pl.kernel,
    out_type=jax.ShapeDtypeStruct((E,), jnp.float32),
    mesh=_mesh,
    compiler_params=pltpu.CompilerParams(needs_layout_passes=False),
    scratch_types=[
        pltpu.VMEM((NPAD,), jnp.float32),
        pltpu.VMEM((EW,), jnp.int32),
        pltpu.VMEM((EW,), jnp.int32),
        pltpu.VMEM((EW,), jnp.float32),
        pltpu.VMEM((EW,), jnp.float32),
    ],
)
def _lap(row_hbm, col_hbm, attr_hbm, dinv_hbm, lap_hbm, dinv_v, rows, cols, attrs, lap_v):
    c = lax.axis_index("c")
    s = lax.axis_index("s")
    gw = c * NS + s
    base = gw * EW
    pltpu.sync_copy(dinv_hbm, dinv_v)
    pltpu.sync_copy(row_hbm.at[pl.ds(base, EW)], rows)
    pltpu.sync_copy(col_hbm.at[pl.ds(base, EW)], cols)
    pltpu.sync_copy(attr_hbm.at[pl.ds(base, EW)], attrs)

    def body(i, carry):
        sl = pl.ds(i * L, L)
        dr = plsc.load_gather(dinv_v, [rows[sl]])
        dc = plsc.load_gather(dinv_v, [cols[sl]])
        lap_v[sl] = -(dr * attrs[sl] * dc)
        return carry

    lax.fori_loop(0, EW // L, body, 0)
    pltpu.sync_copy(lap_v, lap_hbm.at[pl.ds(base, EW)])


CS = 128
NCHW = 78
TAIL = E - NW * NCHW * CS


@functools.partial(
    pl.kernel,
    out_type=jax.ShapeDtypeStruct((NC, NPAD, D), jnp.float32),
    mesh=_mesh,
    compiler_params=pltpu.CompilerParams(needs_layout_passes=False),
    scratch_types=[
        pltpu.VMEM_SHARED((NPAD, D), jnp.float32),
        pltpu.VMEM((CS,), jnp.int32),
        pltpu.VMEM((CS,), jnp.int32),
        pltpu.VMEM((CS,), jnp.int32),
        pltpu.VMEM((CS,), jnp.int32),
        pltpu.VMEM((CS,), jnp.float32),
        pltpu.VMEM((CS,), jnp.float32),
        pltpu.VMEM((CS, D), jnp.float32),
        pltpu.VMEM((CS, D), jnp.float32),
        pltpu.SemaphoreType.DMA,
        pltpu.SemaphoreType.DMA,
        pltpu.SemaphoreType.DMA,
        pltpu.SemaphoreType.DMA,
        pltpu.SemaphoreType.DMA,
        pltpu.SemaphoreType.DMA,
        pltpu.SemaphoreType.DMA,
        pltpu.SemaphoreType.DMA,
        pltpu.SemaphoreType.DMA,
        pltpu.SemaphoreType.DMA,
        pltpu.SemaphoreType.DMA,
        pltpu.SemaphoreType.DMA,
    ],
)
def _spmm(m_hbm, col_hbm, row_hbm, lap_hbm, out_hbm, acc,
          colp0, colp1, rowp0, rowp1, lapp0, lapp1, rbuf0, rbuf1,
          csem0, csem1, gsem0, gsem1, ssem0, ssem1,
          psem0, psem1, qsem0, qsem1, tsem0, tsem1):
    c = lax.axis_index("c")
    s = lax.axis_index("s")
    gw = c * NS + s
    ebase = gw * NCHW * CS

    colp = (colp0, colp1)
    rowp = (rowp0, rowp1)
    lapp = (lapp0, lapp1)
    rbuf = (rbuf0, rbuf1)
    csem = (csem0, csem1)
    gsem = (gsem0, gsem1)
    ssem = (ssem0, ssem1)
    psem = (psem0, psem1)
    qsem = (qsem0, qsem1)

    def zfill(i, carry):
        for j in range(D // L):
            rbuf0[i, pl.ds(j * L, L)] = jnp.zeros((L,), jnp.float32)
        return carry

    lax.fori_loop(0, CS, zfill, 0)
    for k in range(RPW // CS):
        pltpu.sync_copy(rbuf0, acc.at[pl.ds(s * RPW + k * CS, CS), :])
    plsc.subcore_barrier()

    def c_copy(k, b):
        return pltpu.make_async_copy(
            col_hbm.at[pl.ds(ebase + k * CS, CS)], colp[b], csem[b])

    def r_copy(k, b):
        return pltpu.make_async_copy(
            row_hbm.at[pl.ds(ebase + k * CS, CS)], rowp[b], psem[b])

    def l_copy(k, b):
        return pltpu.make_async_copy(
            lap_hbm.at[pl.ds(ebase + k * CS, CS)], lapp[b], qsem[b])

    def g_copy(b):
        return pltpu.make_async_copy(m_hbm.at[colp[b]], rbuf[b], gsem[b])

    def s_copy(b):
        return pltpu.make_async_copy(rbuf[b], acc.at[rowp[b]], ssem[b])

    def scale(b):
        buf = rbuf[b]
        lp = lapp[b]

        def edge(i, carry):
            e = 2 * i
            lv0 = plsc.load_gather(lp, [jnp.zeros((L,), jnp.int32) + e])
            lv1 = plsc.load_gather(lp, [jnp.zeros((L,), jnp.int32) + (e + 1)])
            for j in range(D // L):
                sl = pl.ds(j * L, L)
                buf[e, sl] = buf[e, sl] * lv0
            for j in range(D // L):
                sl = pl.ds(j * L, L)
                buf[e + 1, sl] = buf[e + 1, sl] * lv1
            return carry

        lax.fori_loop(0, CS // 2, edge, 0)

    c_copy(0, 0).start()
    c_copy(1, 1).start()
    r_copy(0, 0).start()
    l_copy(0, 0).start()
    r_copy(1, 1).start()
    l_copy(1, 1).start()
    c_copy(0, 0).wait()
    g_copy(0).start()
    g_copy(0).wait()
    c_copy(2, 0).start()
    r_copy(0, 0).wait()
    l_copy(0, 0).wait()
    c_copy(1, 1).wait()
    g_copy(1).start()
    scale(0)
    s_copy(0).start(add=True)

    def step(k, b):
        ob = 1 - b
        g_copy(b).wait()

        @pl.when(k + 2 < NCHW)
        def _():
            c_copy(k + 2, b).start()

        r_copy(k, b).wait()
        l_copy(k, b).wait()
        s_copy(ob).wait()

        @pl.when(k + 1 < NCHW)
        def _():
            c_copy(k + 1, ob).wait()
            g_copy(ob).start()
            r_copy(k + 1, ob).start()
            l_copy(k + 1, ob).start()

        scale(b)
        s_copy(b).start(add=True)

    def outer(g, carry):
        step(2 * g + 1, 1)
        step(2 * g + 2, 0)
        return carry

    lax.fori_loop(0, (NCHW - 2) // 2, outer, 0)
    step(NCHW - 1, 1)
    s_copy(1).wait()

    @pl.when(gw * CS < TAIL)
    def _():
        tb = (NW * NCHW + gw) * CS
        pltpu.sync_copy(col_hbm.at[pl.ds(tb, CS)], colp0)
        pltpu.sync_copy(row_hbm.at[pl.ds(tb, CS)], rowp0)
        pltpu.sync_copy(lap_hbm.at[pl.ds(tb, CS)], lapp0)
        pltpu.async_copy(m_hbm.at[colp0], rbuf0, tsem0).wait()
        scale(0)
        pltpu.async_copy(rbuf0, acc.at[rowp0], tsem1, add=True).wait()

    plsc.subcore_barrier()
    pltpu.sync_copy(acc.at[pl.ds(s * RPW, RPW), :], out_hbm.at[c, pl.ds(s * RPW, RPW), :])


_RB = 400


def _combine_body(a, b, p0_ref, p1_ref, prev_ref, out_ref):
    out_ref[...] = a * (p0_ref[...] + p1_ref[...]) + b * prev_ref[...]


def _combine(p0, p1, prev, a, b):
    grid = NPAD // 512
    bs = pl.BlockSpec((512, D), lambda i: (i, 0))
    return pl.pallas_call(
        functools.partial(_combine_body, a, b),
        grid=(grid,),
        in_specs=[bs, bs, bs],
        out_specs=bs,
        out_shape=jax.ShapeDtypeStruct((NPAD, D), jnp.float32),
    )(p0, p1, prev)


def _final_body(x_ref, t1_ref, t2_ref, p0_ref, p1_ref, w_ref, b_ref, out_ref):
    t3 = 2.0 * (p0_ref[...] + p1_ref[...]) - t1_ref[...]
    w = w_ref[...]
    acc = jnp.dot(x_ref[...], w[0], preferred_element_type=jnp.float32)
    acc += jnp.dot(t1_ref[...], w[1], preferred_element_type=jnp.float32)
    acc += jnp.dot(t2_ref[...], w[2], preferred_element_type=jnp.float32)
    acc += jnp.dot(t3, w[3], preferred_element_type=jnp.float32)
    out_ref[...] = acc + b_ref[...]


def _final(x, t1, t2, p0, p1, weight, bias):
    grid = N // _RB
    bs = pl.BlockSpec((_RB, D), lambda i: (i, 0))
    return pl.pallas_call(
        _final_body,
        grid=(grid,),
        in_specs=[
            bs, bs, bs, bs, bs,
            pl.BlockSpec((4, D, D), lambda i: (0, 0, 0)),
            pl.BlockSpec((1, D), lambda i: (0, 0)),
        ],
        out_specs=bs,
        out_shape=jax.ShapeDtypeStruct((N, D), jnp.float32),
    )(x, t1, t2, p0, p1, weight, bias.reshape(1, D))


def kernel(x, edge_index, edge_attr, weight, bias):
    row = edge_index[0]
    col = edge_index[1]
    xp = jnp.pad(x, ((0, NPAD - N), (0, 0)))
    degp = _deg(row, edge_attr)
    dinv = _dinv(degp.reshape(NW, NPAD)).reshape(NPAD)
    lap = _lap(row, col, edge_attr, dinv)
    p = _spmm(xp, col, row, lap)
    t1 = _combine(p[0], p[1], xp, 1.0, 0.0)
    p = _spmm(t1, col, row, lap)
    t2 = _combine(p[0], p[1], xp, 2.0, -1.0)
    p = _spmm(t2, col, row, lap)
    return _final(xp, t1, t2, p[0], p[1], weight, bias)


# speedup vs baseline: 16.3259x; 1.0020x over previous
import functools

import jax
import jax.numpy as jnp
from jax import lax
from jax.experimental import pallas as pl
from jax.experimental.pallas import tpu as pltpu
from jax.experimental.pallas import tpu_sc as plsc

N = 10000
E = 320000
D = 128
NPAD = 10240

NC = 2
NS = 16
L = 16
NW = NC * NS
EW = E // NW
C = 80
NCH = EW // C
RPW = NPAD // NS
ZR = 128

_mesh = plsc.VectorSubcoreMesh(core_axis_name="c", subcore_axis_name="s")


@functools.partial(
    pl.kernel,
    out_type=jax.ShapeDtypeStruct((NW * NPAD,), jnp.float32),
    mesh=_mesh,
    compiler_params=pltpu.CompilerParams(needs_layout_passes=False),
    scratch_types=[
        pltpu.VMEM((NPAD,), jnp.float32),
        pltpu.VMEM((EW,), jnp.int32),
        pltpu.VMEM((EW,), jnp.float32),
    ],
)
def _deg(row_hbm, attr_hbm, out_hbm, acc, rows, attrs):
    c = lax.axis_index("c")
    s = lax.axis_index("s")
    gw = c * NS + s
    base = gw * EW
    pltpu.sync_copy(row_hbm.at[pl.ds(base, EW)], rows)
    pltpu.sync_copy(attr_hbm.at[pl.ds(base, EW)], attrs)

    def zero(i, carry):
        acc[pl.ds(i * L, L)] = jnp.zeros((L,), jnp.float32)
        return carry

    lax.fori_loop(0, NPAD // L, zero, 0)

    def body(i, carry):
        r = rows[pl.ds(i * L, L)]
        a = attrs[pl.ds(i * L, L)]
        plsc.addupdate_scatter(acc, [r], a)
        return carry

    lax.fori_loop(0, EW // L, body, 0)
    pltpu.sync_copy(acc, out_hbm.at[pl.ds(gw * NPAD, NPAD)])


def _dinv_body(degp_ref, dinv_ref):
    deg = jnp.sum(degp_ref[...], axis=0)
    r = lax.rsqrt(jnp.maximum(deg, 1e-12))
    dinv_ref[...] = jnp.where(deg > 0, r, 0.0)


def _dinv(degp):
    return pl.pallas_call(
        _dinv_body,
        out_shape=jax.ShapeDtypeStruct((NPAD // 128, 128), jnp.float32),
    )(degp.reshape(NW, NPAD // 128, 128))


@functools.partial(
    pl.kernel,
    out_type=jax.ShapeDtypeStruct((E,), jnp.float32),
    mesh=_mesh,
    compiler_params=pltpu.CompilerParams(needs_layout_passes=False),
    scratch_types=[
        pltpu.VMEM((NPAD,), jnp.float32),
        pltpu.VMEM((EW,), jnp.int32),
        pltpu.VMEM((EW,), jnp.int32),
        pltpu.VMEM((EW,), jnp.float32),
        pltpu.VMEM((EW,), jnp.float32),
    ],
)
def _lap(row_hbm, col_hbm, attr_hbm, dinv_hbm, lap_hbm, dinv_v, rows, cols, attrs, lap_v):
    c = lax.axis_index("c")
    s = lax.axis_index("s")
    gw = c * NS + s
    base = gw * EW
    pltpu.sync_copy(dinv_hbm, dinv_v)
    pltpu.sync_copy(row_hbm.at[pl.ds(base, EW)], rows)
    pltpu.sync_copy(col_hbm.at[pl.ds(base, EW)], cols)
    pltpu.sync_copy(attr_hbm.at[pl.ds(base, EW)], attrs)

    def body(i, carry):
        sl = pl.ds(i * L, L)
        dr = plsc.load_gather(dinv_v, [rows[sl]])
        dc = plsc.load_gather(dinv_v, [cols[sl]])
        lap_v[sl] = -(dr * attrs[sl] * dc)
        return carry

    lax.fori_loop(0, EW // L, body, 0)
    pltpu.sync_copy(lap_v, lap_hbm.at[pl.ds(base, EW)])


CS = 128
NCHW = 78
TAIL = E - NW * NCHW * CS
RW0 = 624
RW1 = N - (NS - 1) * RW0


@functools.partial(
    pl.kernel,
    out_type=jax.ShapeDtypeStruct((NC, N, D), jnp.float32),
    mesh=_mesh,
    compiler_params=pltpu.CompilerParams(needs_layout_passes=False),
    scratch_types=[
        pltpu.VMEM_SHARED((N, D), jnp.float32),
        [pltpu.VMEM((CS,), jnp.int32) for _ in range(2)],
        [pltpu.VMEM((CS,), jnp.int32) for _ in range(3)],
        [pltpu.VMEM((CS,), jnp.float32) for _ in range(2)],
        [pltpu.VMEM((CS, D), jnp.float32) for _ in range(3)],
        [pltpu.SemaphoreType.DMA for _ in range(2)],
        [pltpu.SemaphoreType.DMA for _ in range(3)],
        [pltpu.SemaphoreType.DMA for _ in range(2)],
        [pltpu.SemaphoreType.DMA for _ in range(3)],
        [pltpu.SemaphoreType.DMA for _ in range(3)],
        [pltpu.SemaphoreType.DMA for _ in range(2)],
    ],
)
def _spmm(m_hbm, col_hbm, row_hbm, lap_hbm, out_hbm, acc,
          colp, rowp, lapp, rbuf, csem, psem, qsem, gsem, ssem, tsem):
    c = lax.axis_index("c")
    s = lax.axis_index("s")
    gw = c * NS + s
    ebase = gw * NCHW * CS

    def zfill(i, carry):
        for j in range(D // L):
            rbuf[0][i, pl.ds(j * L, L)] = jnp.zeros((L,), jnp.float32)
        return carry

    lax.fori_loop(0, CS, zfill, 0)

    @pl.when(s < NS - 1)
    def _():
        for k in range(4):
            pltpu.sync_copy(rbuf[0], acc.at[pl.ds(s * RW0 + k * CS, CS), :])
        pltpu.sync_copy(rbuf[0].at[pl.ds(0, RW0 - 4 * CS)],
                        acc.at[pl.ds(s * RW0 + 4 * CS, RW0 - 4 * CS), :])

    @pl.when(s == NS - 1)
    def _():
        for k in range(RW1 // CS):
            pltpu.sync_copy(rbuf[0], acc.at[pl.ds(s * RW0 + k * CS, CS), :])

    plsc.subcore_barrier()

    def c_copy(k, b):
        return pltpu.make_async_copy(
            col_hbm.at[pl.ds(ebase + k * CS, CS)], colp[b], csem[b])

    def r_copy(k, b):
        return pltpu.make_async_copy(
            row_hbm.at[pl.ds(ebase + k * CS, CS)], rowp[b], psem[b])

    def l_copy(k, b):
        return pltpu.make_async_copy(
            lap_hbm.at[pl.ds(ebase + k * CS, CS)], lapp[b], qsem[b])

    def g_copy(b2, b3):
        return pltpu.make_async_copy(m_hbm.at[colp[b2]], rbuf[b3], gsem[b3])

    def s_copy(b3):
        return pltpu.make_async_copy(rbuf[b3], acc.at[rowp[b3]], ssem[b3])

    def scale(b3, b2):
        buf = rbuf[b3]
        lp = lapp[b2]

        def edge(i, carry):
            e = 2 * i
            lv0 = plsc.load_gather(lp, [jnp.zeros((L,), jnp.int32) + e])
            lv1 = plsc.load_gather(lp, [jnp.zeros((L,), jnp.int32) + (e + 1)])
            for j in range(D // L):
                sl = pl.ds(j * L, L)
                buf[e, sl] = buf[e, sl] * lv0
            for j in range(D // L):
                sl = pl.ds(j * L, L)
                buf[e + 1, sl] = buf[e + 1, sl] * lv1
            return carry

        lax.fori_loop(0, CS // 2, edge, 0)

    def step(k, u):
        b3, b2 = u % 3, u % 2
        first = u < 2
        last1 = u >= NCHW - 1
        last2 = u >= NCHW - 2
        g_copy(b2, b3).wait()
        if not last2:
            c_copy(k + 2, b2).start()
        r_copy(k, b3).wait()
        l_copy(k, b2).wait()
        if not first:
            s_copy((u + 1) % 3).wait()
        if not last1:
            c_copy(k + 1, (u + 1) % 2).wait()
            g_copy((u + 1) % 2, (u + 1) % 3).start()
            r_copy(k + 1, (u + 1) % 3).start()
            l_copy(k + 1, (u + 1) % 2).start()
        scale(b3, b2)
        s_copy(b3).start(add=True)

    c_copy(0, 0).start()
    c_copy(1, 1).start()
    r_copy(0, 0).start()
    l_copy(0, 0).start()
    c_copy(0, 0).wait()
    g_copy(0, 0).start()
    step(0, 0)
    step(1, 1)

    def outer(g, carry):
        for u in range(6):
            step(2 + 6 * g + u, 2 + u)
        return carry

    lax.fori_loop(0, 12, outer, 0)
    for k in range(74, NCHW):
        step(k, k)
    s_copy((NCHW - 2) % 3).wait()
    s_copy((NCHW - 1) % 3).wait()

    @pl.when(gw * CS < TAIL)
    def _():
        tb = (NW * NCHW + gw) * CS
        pltpu.sync_copy(col_hbm.at[pl.ds(tb, CS)], colp[0])
        pltpu.sync_copy(row_hbm.at[pl.ds(tb, CS)], rowp[0])
        pltpu.sync_copy(lap_hbm.at[pl.ds(tb, CS)], lapp[0])
        pltpu.async_copy(m_hbm.at[colp[0]], rbuf[0], tsem[0]).wait()
        scale(0, 0)
        pltpu.async_copy(rbuf[0], acc.at[rowp[0]], tsem[1], add=True).wait()

    plsc.subcore_barrier()

    @pl.when(s < NS - 1)
    def _():
        pltpu.sync_copy(acc.at[pl.ds(s * RW0, RW0), :],
                        out_hbm.at[c, pl.ds(s * RW0, RW0), :])

    @pl.when(s == NS - 1)
    def _():
        pltpu.sync_copy(acc.at[pl.ds(s * RW0, RW1), :],
                        out_hbm.at[c, pl.ds(s * RW0, RW1), :])


_RB = 400


def _combine_body(a, b, p0_ref, p1_ref, prev_ref, out_ref):
    out_ref[...] = a * (p0_ref[...] + p1_ref[...]) + b * prev_ref[...]


def _combine(p0, p1, prev, a, b):
    grid = N // _RB
    bs = pl.BlockSpec((_RB, D), lambda i: (i, 0))
    return pl.pallas_call(
        functools.partial(_combine_body, a, b),
        grid=(grid,),
        in_specs=[bs, bs, bs],
        out_specs=bs,
        out_shape=jax.ShapeDtypeStruct((N, D), jnp.float32),
    )(p0, p1, prev)


def _final_body(x_ref, t1_ref, t2_ref, p0_ref, p1_ref, w_ref, b_ref, out_ref):
    t3 = 2.0 * (p0_ref[...] + p1_ref[...]) - t1_ref[...]
    w = w_ref[...]
    acc = jnp.dot(x_ref[...], w[0], preferred_element_type=jnp.float32)
    acc += jnp.dot(t1_ref[...], w[1], preferred_element_type=jnp.float32)
    acc += jnp.dot(t2_ref[...], w[2], preferred_element_type=jnp.float32)
    acc += jnp.dot(t3, w[3], preferred_element_type=jnp.float32)
    out_ref[...] = acc + b_ref[...]


def _final(x, t1, t2, p0, p1, weight, bias):
    grid = N // _RB
    bs = pl.BlockSpec((_RB, D), lambda i: (i, 0))
    return pl.pallas_call(
        _final_body,
        grid=(grid,),
        in_specs=[
            bs, bs, bs, bs, bs,
            pl.BlockSpec((4, D, D), lambda i: (0, 0, 0)),
            pl.BlockSpec((1, D), lambda i: (0, 0)),
        ],
        out_specs=bs,
        out_shape=jax.ShapeDtypeStruct((N, D), jnp.float32),
    )(x, t1, t2, p0, p1, weight, bias.reshape(1, D))


def kernel(x, edge_index, edge_attr, weight, bias):
    row = edge_index[0]
    col = edge_index[1]
    degp = _deg(row, edge_attr)
    dinv = _dinv(degp.reshape(NW, NPAD)).reshape(NPAD)
    lap = _lap(row, col, edge_attr, dinv)
    p = _spmm(x, col, row, lap)
    t1 = _combine(p[0], p[1], x, 1.0, 0.0)
    p = _spmm(t1, col, row, lap)
    t2 = _combine(p[0], p[1], x, 2.0, -1.0)
    p = _spmm(t2, col, row, lap)
    return _final(x, t1, t2, p[0], p[1], weight, bias)
